# baseline (speedup 1.0000x reference)
import functools
import math

import jax
import jax.numpy as jnp
from jax.experimental import pallas as pl


def _body(x_ref, pos_ref, out_ref, *, scale, bs):
    i = pl.program_id(0)
    out_ref[...] = x_ref[...] * scale + pos_ref[pl.ds(i * bs, bs), :][None, :, :]


def kernel(x, pos_table):
    B, S, D = x.shape
    scale = math.sqrt(D)
    BS = 1024
    n_seq = S // BS

    grid = (n_seq, B)
    out = pl.pallas_call(
        functools.partial(_body, scale=scale, bs=BS),
        grid=grid,
        in_specs=[
            pl.BlockSpec((1, BS, D), lambda i, b: (b, i, 0)),
            pl.BlockSpec((S, D), lambda i, b: (0, 0)),
        ],
        out_specs=pl.BlockSpec((1, BS, D), lambda i, b: (b, i, 0)),
        out_shape=jax.ShapeDtypeStruct((B, S, D), x.dtype),
    )(x, pos_table[:S])
    return out


# speedup vs baseline: 1.0287x; 1.0287x over previous
import functools
import math

import jax
import jax.numpy as jnp
from jax.experimental import pallas as pl


def _body(x_ref, pos_ref, out_ref, *, scale, bs):
    i = pl.program_id(0)
    out_ref[...] = x_ref[...] * scale + pos_ref[pl.ds(i * bs, bs), :][None, :, :]


def kernel(x, pos_table):
    B, S, D = x.shape
    scale = math.sqrt(D)
    BS = 2048
    n_seq = S // BS

    grid = (n_seq, B)
    out = pl.pallas_call(
        functools.partial(_body, scale=scale, bs=BS),
        grid=grid,
        in_specs=[
            pl.BlockSpec((1, BS, D), lambda i, b: (b, i, 0)),
            pl.BlockSpec((S, D), lambda i, b: (0, 0)),
        ],
        out_specs=pl.BlockSpec((1, BS, D), lambda i, b: (b, i, 0)),
        out_shape=jax.ShapeDtypeStruct((B, S, D), x.dtype),
    )(x, pos_table[:S])
    return out
